# baseline (device time: 47195 ns/iter reference)
import os

import jax
import jax.numpy as jnp
from jax import lax
from jax.experimental import pallas as pl
from jax.experimental.pallas import tpu as pltpu

SQ = 1024
SKV_SHARD = 1024
HQ = 8
DH = 128
D = HQ * DH
WIN = 128
SCALE = 0.08838834764831843

QBLK = int(os.environ.get("SCBAND_QBLK", "128"))
NBLK = SQ // QBLK
KBAND = QBLK + 256
SLIVER = 128
KV_USED = SKV_SHARD + SLIVER

_KSTART = tuple(
    min(max(QBLK * b - 128, 0), KV_USED - KBAND) for b in range(NBLK))

_L = tuple(range(NBLK // 2))
_R = tuple(range(NBLK // 2, NBLK))
_NBYPASS = max(NBLK // 4, 1)
_BYP_L = _L[-_NBYPASS:]
_BYP_R = _R[-_NBYPASS:]
_ORDER = tuple(b for pair in zip(_L, _R) for b in pair)
_DESTS = {}
for _b in _L:
    _DESTS[_b] = (1, 3) if _b in _BYP_L else (1,)
for _b in _R:
    _DESTS[_b] = (3, 1) if _b in _BYP_R else (3,)
_D1_DIRECT = tuple(b for b in _ORDER if 1 in _DESTS[b])
_D1_RING = tuple(b for b in _R if b not in _BYP_R)
_D3_DIRECT = tuple(b for b in _ORDER if 3 in _DESTS[b])
_D3_RING = tuple(b for b in _L if b not in _BYP_L)
_D2_RELAY = _D3_RING + _D1_RING

BF16 = jnp.bfloat16
ABLATE = os.environ.get("SCBAND_ABLATE", "")


def _mm(a, b):
    return lax.dot_general(a, b, (((1,), (0,)), ((), ())),
                           preferred_element_type=jnp.float32)


def kernel(x, Wq, K_ext, V_ext, Wo):
    x2 = x.reshape(SQ, D).astype(BF16)
    wq16 = Wq.astype(BF16)
    wo16 = Wo.astype(BF16)
    k3 = K_ext.reshape(SKV_SHARD, D).astype(BF16)
    v3 = V_ext.reshape(SKV_SHARD, D).astype(BF16)

    def body(x_ref, wq_ref, k_ref, v_ref, wo_ref, out_ref,
             qbuf, kbuf, vbuf, ctxbuf, obuf, kvsliv,
             sliver_send_sems, sliver_recv_sems,
             send_sems, recv_sems, relay1_sems, relay2_sems):
        my = lax.axis_index("i")

        def blk(ref, b):
            return ref.at[pl.ds(b * QBLK, QBLK), :]

        def compute_all(emit):
            kbuf[pl.ds(0, SKV_SHARD), :] = k_ref[...]
            vbuf[pl.ds(0, SKV_SHARD), :] = v_ref[...]
            qbuf[...] = (_mm(x_ref[...], wq_ref[...]) * SCALE).astype(BF16)

            for b in _ORDER:
                ks = _KSTART[b]
                if ks + KBAND > SKV_SHARD:
                    rd = pltpu.make_async_remote_copy(
                        src_ref=kvsliv,
                        dst_ref=kvsliv,
                        send_sem=sliver_send_sems.at[0],
                        recv_sem=sliver_recv_sems.at[0],
                        device_id=(1,),
                        device_id_type=pl.DeviceIdType.MESH,
                    )
                    rd.wait_recv()
                    kbuf[pl.ds(SKV_SHARD, SLIVER), :] = kvsliv[0]
                    vbuf[pl.ds(SKV_SHARD, SLIVER), :] = kvsliv[1]

                qi = b * QBLK + lax.broadcasted_iota(jnp.int32, (QBLK, KBAND), 0)
                kj = ks + lax.broadcasted_iota(jnp.int32, (QBLK, KBAND), 1)
                mask = jnp.abs(qi - kj) <= WIN

                for h in range(HQ):
                    q_h = qbuf[pl.ds(b * QBLK, QBLK), pl.ds(h * DH, DH)]
                    k_h = kbuf[pl.ds(ks, KBAND), pl.ds(h * DH, DH)]
                    v_h = vbuf[pl.ds(ks, KBAND), pl.ds(h * DH, DH)]
                    s = lax.dot_general(
                        q_h, k_h, (((1,), (1,)), ((), ())),
                        preferred_element_type=jnp.float32)
                    p = jnp.exp(jnp.where(mask, s, -1e30))
                    l = jnp.sum(p, axis=1, keepdims=True)
                    ctx_h = lax.dot_general(
                        (p / l).astype(BF16), v_h, (((1,), (0,)), ((), ())),
                        preferred_element_type=jnp.float32)
                    ctxbuf[:, pl.ds(h * DH, DH)] = ctx_h.astype(BF16)

                out_blk = _mm(ctxbuf[...], wo_ref[...])
                out_ref[0, pl.ds(b * QBLK, QBLK), :] = out_blk
                obuf[pl.ds(b * QBLK, QBLK), :] = out_blk.astype(BF16)
                emit(b)

        if ABLATE == "nocomm":
            compute_all(lambda b: None)
            return

        barrier_sem = pltpu.get_barrier_semaphore()
        for p in (lax.rem(my + 1, 4), lax.rem(my + 3, 4)):
            pl.semaphore_signal(barrier_sem, inc=1, device_id=(p,),
                                device_id_type=pl.DeviceIdType.MESH)
        pl.semaphore_wait(barrier_sem, 2)

        @pl.when(my == 1)
        def _():
            kvsliv[0] = k_ref[pl.ds(0, SLIVER), :]
            kvsliv[1] = v_ref[pl.ds(0, SLIVER), :]
            rd = pltpu.make_async_remote_copy(
                src_ref=kvsliv,
                dst_ref=kvsliv,
                send_sem=sliver_send_sems.at[0],
                recv_sem=sliver_recv_sems.at[0],
                device_id=(0,),
                device_id_type=pl.DeviceIdType.MESH,
            )
            rd.start()
            rd.wait_send()

        @pl.when(my == 0)
        def _():
            sends = []

            def emit(b):
                for t, tgt in enumerate(_DESTS[b]):
                    rd = pltpu.make_async_remote_copy(
                        src_ref=blk(obuf, b),
                        dst_ref=blk(obuf, b),
                        send_sem=send_sems.at[b, t],
                        recv_sem=recv_sems.at[b],
                        device_id=(tgt,),
                        device_id_type=pl.DeviceIdType.MESH,
                    )
                    rd.start()
                    sends.append(rd)

            compute_all(emit)
            for rd in sends:
                rd.wait_send()

        def recv_block(b, frm):
            rd = pltpu.make_async_remote_copy(
                src_ref=blk(obuf, b),
                dst_ref=blk(obuf, b),
                send_sem=send_sems.at[b, 0],
                recv_sem=recv_sems.at[b],
                device_id=(frm,),
                device_id_type=pl.DeviceIdType.MESH,
            )
            rd.wait_recv()

        def upcast(b):
            out_ref[0, pl.ds(b * QBLK, QBLK), :] = (
                obuf[pl.ds(b * QBLK, QBLK), :].astype(jnp.float32))

        def relay(b, sems, tgt):
            rd = pltpu.make_async_remote_copy(
                src_ref=blk(obuf, b),
                dst_ref=blk(obuf, b),
                send_sem=sems.at[b],
                recv_sem=recv_sems.at[b],
                device_id=(tgt,),
                device_id_type=pl.DeviceIdType.MESH,
            )
            rd.start()
            return rd

        def side(direct, ring, route):
            def go():
                relays = []
                for b in direct:
                    recv_block(b, 0)
                    if b in route:
                        relays.append(relay(b, relay1_sems, 2))
                    upcast(b)
                for b in ring:
                    recv_block(b, 2)
                    upcast(b)
                for rd in relays:
                    rd.wait_send()
            return go

        pl.when(my == 1)(side(_D1_DIRECT, _D1_RING, _L))
        pl.when(my == 3)(side(_D3_DIRECT, _D3_RING, _R))

        @pl.when(my == 2)
        def _():
            relays = []
            for j in range(NBLK // 2):
                for b, frm, tgt in ((_L[j], 1, 3), (_R[j], 3, 1)):
                    recv_block(b, frm)
                    if b in _D2_RELAY:
                        relays.append(relay(b, relay2_sems, tgt))
                    upcast(b)
            for rd in relays:
                rd.wait_send()

    out = pl.pallas_call(
        body,
        out_shape=jax.ShapeDtypeStruct((1, SQ, D), jnp.float32),
        in_specs=[pl.BlockSpec(memory_space=pltpu.VMEM)] * 5,
        out_specs=pl.BlockSpec(memory_space=pltpu.VMEM),
        scratch_shapes=[
            pltpu.VMEM((SQ, D), BF16),
            pltpu.VMEM((KV_USED, D), BF16),
            pltpu.VMEM((KV_USED, D), BF16),
            pltpu.VMEM((QBLK, D), BF16),
            pltpu.VMEM((SQ, D), BF16),
            pltpu.VMEM((2, SLIVER, D), BF16),
            pltpu.SemaphoreType.DMA((1,)),
            pltpu.SemaphoreType.DMA((1,)),
            pltpu.SemaphoreType.DMA((NBLK, 2)),
            pltpu.SemaphoreType.DMA((NBLK,)),
            pltpu.SemaphoreType.DMA((NBLK,)),
            pltpu.SemaphoreType.DMA((NBLK,)),
        ],
        compiler_params=pltpu.CompilerParams(collective_id=0),
    )(x2, wq16, k3, v3, wo16)

    return out


# device time: 42624 ns/iter; 1.1072x vs baseline; 1.1072x over previous
import os

import jax
import jax.numpy as jnp
from jax import lax
from jax.experimental import pallas as pl
from jax.experimental.pallas import tpu as pltpu

SQ = 1024
SKV_SHARD = 1024
HQ = 8
DH = 128
D = HQ * DH
WIN = 128
SCALE = 0.08838834764831843

QBLK = int(os.environ.get("SCBAND_QBLK", "128"))
NBLK = SQ // QBLK
KBAND = QBLK + 256
SLIVER = 128
KV_USED = SKV_SHARD + SLIVER

_KSTART = tuple(
    min(max(QBLK * b - 128, 0), KV_USED - KBAND) for b in range(NBLK))

_L = tuple(range(NBLK // 2))
_R = tuple(range(NBLK // 2, NBLK))
_NBYPASS = max(NBLK // 4, 1)
_BYP_L = _L[-_NBYPASS:]
_BYP_R = _R[-_NBYPASS:]
_ORDER = tuple(b for pair in zip(_L, _R) for b in pair)
_DESTS = {}
for _b in _L:
    _DESTS[_b] = (1, 3) if _b in _BYP_L else (1,)
for _b in _R:
    _DESTS[_b] = (3, 1) if _b in _BYP_R else (3,)
_D1_DIRECT = tuple(b for b in _ORDER if 1 in _DESTS[b])
_D1_RING = tuple(b for b in _R if b not in _BYP_R)
_D3_DIRECT = tuple(b for b in _ORDER if 3 in _DESTS[b])
_D3_RING = tuple(b for b in _L if b not in _BYP_L)
_D2_RELAY = _D3_RING + _D1_RING

BF16 = jnp.bfloat16
ABLATE = os.environ.get("SCBAND_ABLATE", "")


def _mm(a, b):
    return lax.dot_general(a, b, (((1,), (0,)), ((), ())),
                           preferred_element_type=jnp.float32)


def kernel(x, Wq, K_ext, V_ext, Wo):
    x2 = x.reshape(SQ, D)
    k3 = K_ext.reshape(SKV_SHARD, D).astype(BF16)
    v3 = V_ext.reshape(SKV_SHARD, D).astype(BF16)

    def body(x_ref, wq_ref, k_ref, v_ref, wo_ref, out_ref,
             kb7, vb7, ctxbuf, kvsliv,
             sliver_send_sems, sliver_recv_sems,
             send_sems, recv_sems, relay1_sems, relay2_sems, *obufs):
        my = lax.axis_index("i")

        def compute_all(emit):
            wq16 = wq_ref[...].astype(BF16)
            wo16 = wo_ref[...].astype(BF16)

            for b in _ORDER:
                ks = _KSTART[b]
                tail = ks + KBAND > SKV_SHARD
                if tail:
                    rd = pltpu.make_async_remote_copy(
                        src_ref=kvsliv,
                        dst_ref=kvsliv,
                        send_sem=sliver_send_sems.at[0],
                        recv_sem=sliver_recv_sems.at[0],
                        device_id=(1,),
                        device_id_type=pl.DeviceIdType.MESH,
                    )
                    rd.wait_recv()
                    n1 = SKV_SHARD - ks
                    kb7[pl.ds(0, n1), :] = k_ref[pl.ds(ks, n1), :]
                    vb7[pl.ds(0, n1), :] = v_ref[pl.ds(ks, n1), :]
                    kb7[pl.ds(n1, SLIVER), :] = kvsliv[0]
                    vb7[pl.ds(n1, SLIVER), :] = kvsliv[1]

                q16 = (_mm(x_ref[pl.ds(b * QBLK, QBLK), :].astype(BF16),
                           wq16) * SCALE).astype(BF16)

                qi = b * QBLK + lax.broadcasted_iota(jnp.int32, (QBLK, KBAND), 0)
                kj = ks + lax.broadcasted_iota(jnp.int32, (QBLK, KBAND), 1)
                mask = jnp.abs(qi - kj) <= WIN

                for h in range(HQ):
                    q_h = q16[:, h * DH:(h + 1) * DH]
                    if tail:
                        k_h = kb7[:, pl.ds(h * DH, DH)]
                        v_h = vb7[:, pl.ds(h * DH, DH)]
                    else:
                        k_h = k_ref[pl.ds(ks, KBAND), pl.ds(h * DH, DH)]
                        v_h = v_ref[pl.ds(ks, KBAND), pl.ds(h * DH, DH)]
                    s = lax.dot_general(
                        q_h, k_h, (((1,), (1,)), ((), ())),
                        preferred_element_type=jnp.float32)
                    p = jnp.exp(jnp.where(mask, s, -1e30))
                    l = jnp.sum(p, axis=1, keepdims=True)
                    ctx_h = lax.dot_general(
                        (p / l).astype(BF16), v_h, (((1,), (0,)), ((), ())),
                        preferred_element_type=jnp.float32)
                    ctxbuf[:, pl.ds(h * DH, DH)] = ctx_h.astype(BF16)

                out_blk = _mm(ctxbuf[...], wo16)
                out_ref[0, pl.ds(b * QBLK, QBLK), :] = out_blk
                obufs[b][...] = out_blk.astype(BF16)
                emit(b)

        if ABLATE == "nocomm":
            compute_all(lambda b: None)
            return

        barrier_sem = pltpu.get_barrier_semaphore()
        for p in (lax.rem(my + 1, 4), lax.rem(my + 3, 4)):
            pl.semaphore_signal(barrier_sem, inc=1, device_id=(p,),
                                device_id_type=pl.DeviceIdType.MESH)

        @pl.when(my != 0)
        def _():
            pl.semaphore_wait(barrier_sem, 2)

        @pl.when(my == 1)
        def _():
            kvsliv[0] = k_ref[pl.ds(0, SLIVER), :]
            kvsliv[1] = v_ref[pl.ds(0, SLIVER), :]
            rd = pltpu.make_async_remote_copy(
                src_ref=kvsliv,
                dst_ref=kvsliv,
                send_sem=sliver_send_sems.at[0],
                recv_sem=sliver_recv_sems.at[0],
                device_id=(0,),
                device_id_type=pl.DeviceIdType.MESH,
            )
            rd.start()
            rd.wait_send()

        @pl.when(my == 0)
        def _():
            sends = []
            barrier_done = [False]

            def emit(b):
                if not barrier_done[0]:
                    pl.semaphore_wait(barrier_sem, 2)
                    barrier_done[0] = True
                for t, tgt in enumerate(_DESTS[b]):
                    rd = pltpu.make_async_remote_copy(
                        src_ref=obufs[b],
                        dst_ref=obufs[b],
                        send_sem=send_sems.at[b, t],
                        recv_sem=recv_sems.at[b],
                        device_id=(tgt,),
                        device_id_type=pl.DeviceIdType.MESH,
                    )
                    rd.start()
                    sends.append(rd)

            compute_all(emit)
            for rd in sends:
                rd.wait_send()

        def recv_block(b, frm):
            rd = pltpu.make_async_remote_copy(
                src_ref=obufs[b],
                dst_ref=obufs[b],
                send_sem=send_sems.at[b, 0],
                recv_sem=recv_sems.at[b],
                device_id=(frm,),
                device_id_type=pl.DeviceIdType.MESH,
            )
            rd.wait_recv()

        def upcast(b):
            out_ref[0, pl.ds(b * QBLK, QBLK), :] = obufs[b][...].astype(jnp.float32)

        def relay(b, sems, tgt):
            rd = pltpu.make_async_remote_copy(
                src_ref=obufs[b],
                dst_ref=obufs[b],
                send_sem=sems.at[b],
                recv_sem=recv_sems.at[b],
                device_id=(tgt,),
                device_id_type=pl.DeviceIdType.MESH,
            )
            rd.start()
            return rd

        def side(direct, ring, route):
            def go():
                relays = []
                for b in direct:
                    recv_block(b, 0)
                    if b in route:
                        relays.append(relay(b, relay1_sems, 2))
                    upcast(b)
                for b in ring:
                    recv_block(b, 2)
                    upcast(b)
                for rd in relays:
                    rd.wait_send()
            return go

        pl.when(my == 1)(side(_D1_DIRECT, _D1_RING, _L))
        pl.when(my == 3)(side(_D3_DIRECT, _D3_RING, _R))

        @pl.when(my == 2)
        def _():
            relays = []
            for j in range(NBLK // 2):
                for b, frm, tgt in ((_L[j], 1, 3), (_R[j], 3, 1)):
                    recv_block(b, frm)
                    if b in _D2_RELAY:
                        relays.append(relay(b, relay2_sems, tgt))
                    upcast(b)
            for rd in relays:
                rd.wait_send()

    out = pl.pallas_call(
        body,
        out_shape=jax.ShapeDtypeStruct((1, SQ, D), jnp.float32),
        in_specs=[pl.BlockSpec(memory_space=pltpu.VMEM)] * 5,
        out_specs=pl.BlockSpec(memory_space=pltpu.VMEM),
        scratch_shapes=[
            pltpu.VMEM((KBAND, D), BF16),
            pltpu.VMEM((KBAND, D), BF16),
            pltpu.VMEM((QBLK, D), BF16),
            pltpu.VMEM((2, SLIVER, D), BF16),
            pltpu.SemaphoreType.DMA((1,)),
            pltpu.SemaphoreType.DMA((1,)),
            pltpu.SemaphoreType.DMA((NBLK, 2)),
            pltpu.SemaphoreType.DMA((NBLK,)),
            pltpu.SemaphoreType.DMA((NBLK,)),
            pltpu.SemaphoreType.DMA((NBLK,)),
        ] + [
            pltpu.VMEM((QBLK, D), BF16) for _ in range(NBLK)
        ],
        compiler_params=pltpu.CompilerParams(collective_id=0),
    )(x2, Wq, k3, v3, Wo)

    return out
